# baseline (speedup 1.0000x reference)
import functools

import jax
import jax.numpy as jnp
from jax import lax
from jax.experimental import pallas as pl
from jax.experimental.pallas import tpu as pltpu
from jax.experimental.pallas import tpu_sc as plsc

N_CLASSES = 256
TOTAL_BINS = 8

NUM_CORES = 2
NUM_SUBCORES = 16
NUM_WORKERS = NUM_CORES * NUM_SUBCORES

CHUNK = 2048


def _make_sc_lookup(n_idx: int):
    assert n_idx % (NUM_WORKERS * CHUNK) == 0
    per_w = n_idx // NUM_WORKERS
    n_chunks = per_w // CHUNK
    assert n_chunks % 2 == 0 and n_chunks >= 4
    n_pairs = n_chunks // 2

    mesh = plsc.VectorSubcoreMesh(
        core_axis_name="c", subcore_axis_name="s",
        num_cores=NUM_CORES, num_subcores=NUM_SUBCORES)

    @functools.partial(
        pl.kernel,
        out_type=jax.ShapeDtypeStruct((n_idx, TOTAL_BINS), jnp.float32),
        mesh=mesh,
        scratch_types=[
            pltpu.VMEM((CHUNK,), jnp.int32),
            pltpu.VMEM((CHUNK,), jnp.int32),
            pltpu.VMEM((CHUNK, TOTAL_BINS), jnp.float32),
            pltpu.VMEM((CHUNK, TOTAL_BINS), jnp.float32),
            pltpu.SemaphoreType.DMA,
            pltpu.SemaphoreType.DMA,
            pltpu.SemaphoreType.DMA,
            pltpu.SemaphoreType.DMA,
            pltpu.SemaphoreType.DMA,
            pltpu.SemaphoreType.DMA,
        ],
        compiler_params=pltpu.CompilerParams(use_tc_tiling_on_sc=False),
    )
    def lookup(table_hbm, idx_hbm, out_hbm, ib0, ib1, rb0, rb1,
               si0, si1, sg0, sg1, so0, so1):
        wid = lax.axis_index("s") * NUM_CORES + lax.axis_index("c")
        base = wid * per_w

        bufs = ((ib0, rb0, si0, sg0, so0), (ib1, rb1, si1, sg1, so1))

        def start_i(c, b):
            ib, _, si, _, _ = bufs[b]
            pltpu.async_copy(idx_hbm.at[pl.ds(base + c * CHUNK, CHUNK)],
                             ib, si)

        def stage(c, b, wait_o, next_i):
            ib, rb, si, sg, so = bufs[b]
            off = base + c * CHUNK
            pltpu.make_async_copy(
                idx_hbm.at[pl.ds(off, CHUNK)], ib, si).wait()
            if wait_o:
                pltpu.make_async_copy(
                    rb, out_hbm.at[pl.ds(off, CHUNK)], so).wait()
            pltpu.async_copy(table_hbm.at[ib], rb, sg).wait()
            pltpu.async_copy(rb, out_hbm.at[pl.ds(off, CHUNK)], so)
            if next_i:
                start_i(c + 2, b)

        start_i(0, 0)
        start_i(1, 1)
        stage(0, 0, wait_o=False, next_i=True)
        stage(1, 1, wait_o=False, next_i=True)

        def pair_body(p, carry):
            c0 = 2 * p
            stage(c0, 0, wait_o=True, next_i=True)
            stage(c0 + 1, 1, wait_o=True, next_i=True)
            return carry

        lax.fori_loop(1, n_pairs - 1, pair_body, 0)

        c0 = n_chunks - 2
        stage(c0, 0, wait_o=True, next_i=False)
        stage(c0 + 1, 1, wait_o=True, next_i=False)
        for b in (0, 1):
            ib, rb, si, sg, so = bufs[b]
            pltpu.make_async_copy(
                rb, out_hbm.at[pl.ds(base, CHUNK)], so).wait()

    return lookup


def kernel(codebook, idx):
    n_idx = idx.size
    rows = _make_sc_lookup(n_idx)(codebook, idx.reshape(n_idx))
    return rows.reshape(idx.shape + (2, TOTAL_BINS // 2))


# speedup vs baseline: 1.0006x; 1.0006x over previous
import functools

import jax
import jax.numpy as jnp
from jax import lax
from jax.experimental import pallas as pl
from jax.experimental.pallas import tpu as pltpu
from jax.experimental.pallas import tpu_sc as plsc

N_CLASSES = 256
TOTAL_BINS = 8

NUM_CORES = 2
NUM_SUBCORES = 16
NUM_WORKERS = NUM_CORES * NUM_SUBCORES

CHUNK = 2048


def _make_sc_lookup(n_idx: int):
    assert n_idx % (NUM_WORKERS * CHUNK) == 0
    per_w = n_idx // NUM_WORKERS
    n_chunks = per_w // CHUNK
    assert n_chunks % 2 == 0 and n_chunks >= 4

    mesh = plsc.VectorSubcoreMesh(
        core_axis_name="c", subcore_axis_name="s",
        num_cores=NUM_CORES, num_subcores=NUM_SUBCORES)

    @functools.partial(
        pl.kernel,
        out_type=jax.ShapeDtypeStruct((n_idx, TOTAL_BINS), jnp.float32),
        mesh=mesh,
        scratch_types=[
            pltpu.VMEM((CHUNK,), jnp.int32),
            pltpu.VMEM((CHUNK,), jnp.int32),
            pltpu.VMEM((CHUNK, TOTAL_BINS), jnp.float32),
            pltpu.VMEM((CHUNK, TOTAL_BINS), jnp.float32),
            pltpu.SemaphoreType.DMA,
            pltpu.SemaphoreType.DMA,
            pltpu.SemaphoreType.DMA,
            pltpu.SemaphoreType.DMA,
            pltpu.SemaphoreType.DMA,
            pltpu.SemaphoreType.DMA,
        ],
        compiler_params=pltpu.CompilerParams(use_tc_tiling_on_sc=False),
    )
    def lookup(table_hbm, idx_hbm, out_hbm, ib0, ib1, rb0, rb1,
               si0, si1, sg0, sg1, so0, so1):
        wid = lax.axis_index("s") * NUM_CORES + lax.axis_index("c")
        base = wid * per_w

        bufs = ((ib0, rb0, si0, sg0, so0), (ib1, rb1, si1, sg1, so1))

        def start_i(c, b):
            ib, _, si, _, _ = bufs[b]
            pltpu.async_copy(idx_hbm.at[pl.ds(base + c * CHUNK, CHUNK)],
                             ib, si)

        def step(c, b, wait_o, next_i):
            ib, rb, si, sg, so = bufs[b]
            ibp, rbp, sip, sgp, sop = bufs[1 - b]
            off = base + c * CHUNK
            pltpu.make_async_copy(
                idx_hbm.at[pl.ds(off, CHUNK)], ib, si).wait()
            if wait_o:
                pltpu.make_async_copy(
                    rb, out_hbm.at[pl.ds(off, CHUNK)], so).wait()
            pltpu.async_copy(table_hbm.at[ib], rb, sg)
            pltpu.make_async_copy(table_hbm.at[ibp], rbp, sgp).wait()
            pltpu.async_copy(
                rbp, out_hbm.at[pl.ds(off - CHUNK, CHUNK)], sop)
            if next_i:
                start_i(c + 1, 1 - b)

        start_i(0, 0)
        start_i(1, 1)
        ib, rb, si, sg, _ = bufs[0]
        pltpu.make_async_copy(
            idx_hbm.at[pl.ds(base, CHUNK)], ib, si).wait()
        pltpu.async_copy(table_hbm.at[ib], rb, sg)
        step(1, 1, wait_o=False, next_i=True)

        def pair_body(p, carry):
            c0 = 2 * p
            step(c0, 0, wait_o=True, next_i=True)
            step(c0 + 1, 1, wait_o=True, next_i=True)
            return carry

        lax.fori_loop(1, n_chunks // 2 - 1, pair_body, 0)

        c0 = n_chunks - 2
        step(c0, 0, wait_o=True, next_i=True)
        step(c0 + 1, 1, wait_o=True, next_i=False)
        ib, rb, si, sg, so = bufs[1]
        off = base + (n_chunks - 1) * CHUNK
        pltpu.make_async_copy(table_hbm.at[ib], rb, sg).wait()
        pltpu.async_copy(rb, out_hbm.at[pl.ds(off, CHUNK)], so)
        for b in (0, 1):
            _, rb, _, _, so = bufs[b]
            pltpu.make_async_copy(
                rb, out_hbm.at[pl.ds(base, CHUNK)], so).wait()

    return lookup


def kernel(codebook, idx):
    n_idx = idx.size
    rows = _make_sc_lookup(n_idx)(codebook, idx.reshape(n_idx))
    return rows.reshape(idx.shape + (2, TOTAL_BINS // 2))


# speedup vs baseline: 1.0020x; 1.0014x over previous
import functools

import jax
import jax.numpy as jnp
from jax import lax
from jax.experimental import pallas as pl
from jax.experimental.pallas import tpu as pltpu
from jax.experimental.pallas import tpu_sc as plsc

N_CLASSES = 256
TOTAL_BINS = 8

NUM_CORES = 2
NUM_SUBCORES = 16
NUM_WORKERS = NUM_CORES * NUM_SUBCORES

IDX_MINOR = 128
ROWS = 16
CHUNK = ROWS * IDX_MINOR


def _make_sc_lookup(n_idx: int):
    assert n_idx % (NUM_WORKERS * CHUNK) == 0
    per_w = n_idx // NUM_WORKERS
    n_chunks = per_w // CHUNK

    mesh = plsc.VectorSubcoreMesh(
        core_axis_name="c", subcore_axis_name="s",
        num_cores=NUM_CORES, num_subcores=NUM_SUBCORES)

    @functools.partial(
        pl.kernel,
        out_type=jax.ShapeDtypeStruct((n_idx, TOTAL_BINS), jnp.float32),
        mesh=mesh,
        scratch_types=[
            pltpu.VMEM((ROWS, IDX_MINOR), jnp.int32),
            pltpu.VMEM((CHUNK, TOTAL_BINS), jnp.float32),
            pltpu.SemaphoreType.DMA,
        ],
        compiler_params=pltpu.CompilerParams(use_tc_tiling_on_sc=False),
    )
    def lookup(table_hbm, idx_hbm, out_hbm, idx_v, rows_v, sem):
        wid = lax.axis_index("s") * NUM_CORES + lax.axis_index("c")
        base_row = wid * (per_w // IDX_MINOR)

        def chunk_body(ci, carry):
            row0 = base_row + ci * ROWS
            pltpu.sync_copy(idx_hbm.at[pl.ds(row0, ROWS)], idx_v)
            copies = []
            for b in range(ROWS):
                copies.append(pltpu.async_copy(
                    table_hbm.at[idx_v.at[b]],
                    rows_v.at[pl.ds(b * IDX_MINOR, IDX_MINOR)],
                    sem))
            for c in copies:
                c.wait()
            off = (base_row + ci * ROWS) * IDX_MINOR
            pltpu.sync_copy(rows_v, out_hbm.at[pl.ds(off, CHUNK)])
            return carry

        lax.fori_loop(0, n_chunks, chunk_body, 0)

    return lookup


def kernel(codebook, idx):
    n_idx = idx.size
    idx2d = idx.reshape(n_idx // IDX_MINOR, IDX_MINOR)
    rows = _make_sc_lookup(n_idx)(codebook, idx2d)
    return rows.reshape(idx.shape + (2, TOTAL_BINS // 2))
